# Initial kernel scaffold; baseline (speedup 1.0000x reference)
import jax
import jax.numpy as jnp
from jax.experimental import pallas as pl


def kernel(edge_index, edge_weight, user_emb, item_emb):
    raise NotImplementedError("write your pallas kernel here")


# speedup vs baseline: 6.5647x; 6.5647x over previous
import functools

import jax
import jax.numpy as jnp
from jax import lax
from jax.experimental import pallas as pl
from jax.experimental.pallas import tpu as pltpu
from jax.experimental.pallas import tpu_sc as plsc

NU = 25000
NI = 25000
D = 64
E = 800000
P = 25088
PADOFF = P - NU
NPAD = 2 * P
SL = NPAD // 16
ROWS_T = P // 16
E_PAD = 804864
ET = E_PAD // 16
EW = E_PAD // 32
CH = 6288
K = 384
NCHUNK = ET // K
_MESH = plsc.VectorSubcoreMesh(core_axis_name="c", subcore_axis_name="s")


def _rsqrt16(d):
    i = lax.bitcast_convert_type(d, jnp.int32)
    i = jnp.full((16,), 0x5F3759DF, jnp.int32) - lax.shift_right_logical(i, 1)
    y = lax.bitcast_convert_type(i, jnp.float32)
    for _ in range(3):
        y = y * (1.5 - 0.5 * d * y * y)
    return y


def _adj16(v):
    off = jnp.full((16,), PADOFF, jnp.int32)
    return v + jnp.where(v >= NU, off, jnp.zeros((16,), jnp.int32))


@functools.partial(
    pl.kernel,
    out_type=jax.ShapeDtypeStruct((E_PAD,), jnp.float32),
    mesh=_MESH,
    compiler_params=pltpu.CompilerParams(needs_layout_passes=False, use_tc_tiling_on_sc=False),
    scratch_types=[
        pltpu.VMEM((NPAD,), jnp.float32),
        pltpu.VMEM((CH,), jnp.int32),
        pltpu.VMEM((CH,), jnp.int32),
        pltpu.VMEM((CH,), jnp.float32),
        pltpu.VMEM((CH,), jnp.float32),
        pltpu.VMEM((SL,), jnp.float32),
        pltpu.VMEM_SHARED((16 * NPAD,), jnp.float32),
    ],
)
def _prep(row_hbm, col_hbm, w_hbm, norm_hbm, dloc, colb, rowb, wb, nb, tmpb,
          shared):
    c = lax.axis_index("c")
    s = lax.axis_index("s")
    zero16 = jnp.zeros((16,), jnp.float32)

    def zbody(i, carry):
        dloc[pl.ds(i * 16, 16)] = zero16
        return carry

    lax.fori_loop(0, NPAD // 16, zbody, 0)

    base_t = s * ET
    one16 = jnp.ones((16,), jnp.float32)
    for ch in range(8):
        off = base_t + ch * CH
        pltpu.sync_copy(col_hbm.at[pl.ds(off, CH)], colb)

        def dbody(g, carry):
            c16 = colb[pl.ds(g * 16, 16)]
            gi = off + g * 16 + lax.iota(jnp.int32, 16)
            ones = jnp.where(gi < E, one16, zero16)
            plsc.addupdate_scatter(dloc, [_adj16(c16)], ones)
            return carry

        lax.fori_loop(0, CH // 16, dbody, 0)

    pltpu.sync_copy(dloc, shared.at[pl.ds(s * NPAD, NPAD)])
    plsc.subcore_barrier()

    pltpu.sync_copy(shared.at[pl.ds(s * SL, SL)], dloc.at[pl.ds(0, SL)])
    for k in range(1, 16):
        pltpu.sync_copy(shared.at[pl.ds(k * NPAD + s * SL, SL)], tmpb)

        def abody(i, carry):
            sl = pl.ds(i * 16, 16)
            dloc[sl] = dloc[sl] + tmpb[sl]
            return carry

        lax.fori_loop(0, SL // 16, abody, 0)
    plsc.subcore_barrier()

    def rbody(i, carry):
        d = dloc[pl.ds(i * 16, 16)]
        y = _rsqrt16(d)
        dloc[pl.ds(i * 16, 16)] = jnp.where(d > 0.0, y, zero16)
        return carry

    lax.fori_loop(0, SL // 16, rbody, 0)
    pltpu.sync_copy(dloc.at[pl.ds(0, SL)], shared.at[pl.ds(s * SL, SL)])
    plsc.subcore_barrier()
    pltpu.sync_copy(shared.at[pl.ds(0, NPAD)], dloc)

    base_n = (c * 16 + s) * EW
    for ch in range(4):
        off = base_n + ch * CH
        pltpu.sync_copy(row_hbm.at[pl.ds(off, CH)], rowb)
        pltpu.sync_copy(col_hbm.at[pl.ds(off, CH)], colb)
        pltpu.sync_copy(w_hbm.at[pl.ds(off, CH)], wb)

        def nbody(g, carry):
            r16 = rowb[pl.ds(g * 16, 16)]
            c16 = colb[pl.ds(g * 16, 16)]
            a = plsc.load_gather(dloc, [_adj16(r16)])
            b = plsc.load_gather(dloc, [_adj16(c16)])
            nb[pl.ds(g * 16, 16)] = a * b * wb[pl.ds(g * 16, 16)]
            return carry

        lax.fori_loop(0, CH // 16, nbody, 0)
        pltpu.sync_copy(nb, norm_hbm.at[pl.ds(off, CH)])


@functools.partial(
    pl.kernel,
    out_type=jax.ShapeDtypeStruct((NPAD, D), jnp.float32),
    mesh=_MESH,
    compiler_params=pltpu.CompilerParams(needs_layout_passes=False, use_tc_tiling_on_sc=False),
    scratch_types=[
        pltpu.VMEM((K, D), jnp.float32),
        pltpu.VMEM((3, 128), jnp.int32),
        pltpu.VMEM((3, 128), jnp.int32),
        pltpu.VMEM((K,), jnp.int32),
        pltpu.VMEM((K,), jnp.int32),
        pltpu.VMEM((K,), jnp.float32),
        pltpu.VMEM_SHARED((P, D), jnp.float32),
        pltpu.SemaphoreType.DMA,
    ],
)
def _layer(x_hbm, row_hbm, col_hbm, norm_hbm, out_hbm,
           rows, ridx, cidx, rowi, coli, normi, acc, sem):
    c = lax.axis_index("c")
    s = lax.axis_index("s")
    zero16 = jnp.zeros((16,), jnp.float32)

    def zbody(r, carry):
        for q in range(4):
            rows[r, pl.ds(q * 16, 16)] = zero16
        return carry

    lax.fori_loop(0, K, zbody, 0)
    arow = s * ROWS_T
    for off, sz in ((0, 384), (384, 384), (768, 384), (1152, 384), (1536, 32)):
        pltpu.sync_copy(rows.at[pl.ds(0, sz)], acc.at[pl.ds(arow + off, sz)])
    plsc.subcore_barrier()

    base_t = s * ET
    cP = c * P
    loc_lo = jnp.zeros((16,), jnp.int32)
    loc_hi = jnp.full((16,), P, jnp.int32)

    def chunk(t, carry):
        off = base_t + t * K
        pltpu.sync_copy(row_hbm.at[pl.ds(off, K)], rowi)
        pltpu.sync_copy(col_hbm.at[pl.ds(off, K)], coli)
        pltpu.sync_copy(norm_hbm.at[pl.ds(off, K)], normi)
        descs = []
        for j in range(3):

            def gb(g, carry2, j=j):
                r16 = rowi[pl.ds(j * 128 + g * 16, 16)]
                ridx[j, pl.ds(g * 16, 16)] = _adj16(r16)
                return carry2

            lax.fori_loop(0, 8, gb, 0)
            descs.append(pltpu.async_copy(
                x_hbm.at[ridx.at[j]], rows.at[pl.ds(j * 128, 128)], sem))
        for j in range(3):

            def cb(g, carry2, j=j):
                bg = j * 128 + g * 16
                c16 = coli[pl.ds(bg, 16)]
                loc = _adj16(c16) - cP
                m = (loc >= loc_lo) & (loc < loc_hi)
                cidx[j, pl.ds(g * 16, 16)] = jnp.where(m, loc, loc_lo)
                n16 = normi[pl.ds(bg, 16)]
                normi[pl.ds(bg, 16)] = jnp.where(m, n16, zero16)
                return carry2

            lax.fori_loop(0, 8, cb, 0)
        for d in descs:
            d.wait()

        def sb(r, carry2):
            sp = plsc.load_gather(normi, [lax.broadcast(r, (16,))])
            for q in range(4):
                rows[r, pl.ds(q * 16, 16)] = rows[r, pl.ds(q * 16, 16)] * sp
            return carry2

        lax.fori_loop(0, K, sb, 0)
        for j in range(3):
            pltpu.sync_copy(rows.at[pl.ds(j * 128, 128)],
                            acc.at[cidx.at[j]], add=True)
        return carry

    lax.fori_loop(0, NCHUNK, chunk, 0)
    plsc.subcore_barrier()
    for off, sz in ((0, 384), (384, 384), (768, 384), (1152, 384), (1536, 32)):
        pltpu.sync_copy(acc.at[pl.ds(arow + off, sz)],
                        out_hbm.at[pl.ds(cP + arow + off, sz)])


def _mean_body(a_ref, b_ref, c_ref, d_ref, o_ref):
    o_ref[...] = 0.25 * (a_ref[...] + b_ref[...] + c_ref[...] + d_ref[...])


_BR = (NPAD * D // 128) // 8
_mean4 = pl.pallas_call(
    _mean_body,
    grid=(8,),
    in_specs=[pl.BlockSpec((_BR, 128), lambda i: (i, 0))] * 4,
    out_specs=pl.BlockSpec((_BR, 128), lambda i: (i, 0)),
    out_shape=jax.ShapeDtypeStruct((NPAD * D // 128, 128), jnp.float32),
)


def kernel(edge_index, edge_weight, user_emb, item_emb):
    row = jnp.pad(edge_index[0], (0, E_PAD - E))
    col = jnp.pad(edge_index[1], (0, E_PAD - E))
    w = jnp.pad(edge_weight, (0, E_PAD - E))
    x0 = jnp.zeros((NPAD, D), jnp.float32)
    x0 = lax.dynamic_update_slice(x0, user_emb, (0, 0))
    x0 = lax.dynamic_update_slice(x0, item_emb, (P, 0))

    norm = _prep(row, col, w)
    x1 = _layer(x0, row, col, norm)
    x2 = _layer(x1, row, col, norm)
    x3 = _layer(x2, row, col, norm)

    flat = lambda a: a.reshape(NPAD * D // 128, 128)
    final = _mean4(flat(x0), flat(x1), flat(x2), flat(x3)).reshape(NPAD, D)
    return final[:NU], final[P:P + NI]


# speedup vs baseline: 8.5529x; 1.3029x over previous
import functools

import jax
import jax.numpy as jnp
from jax import lax
from jax.experimental import pallas as pl
from jax.experimental.pallas import tpu as pltpu
from jax.experimental.pallas import tpu_sc as plsc

NU = 25000
NI = 25000
D = 64
E = 800000
P = 25088
PADOFF = P - NU
NPAD = 2 * P
SL = NPAD // 16
ROWS_T = P // 16
E_PAD = 804864
ET = E_PAD // 16
EW = E_PAD // 32
CH = 6288
K = 384
NCHUNK = ET // K
_MESH = plsc.VectorSubcoreMesh(core_axis_name="c", subcore_axis_name="s")


def _rsqrt16(d):
    i = lax.bitcast_convert_type(d, jnp.int32)
    i = jnp.full((16,), 0x5F3759DF, jnp.int32) - lax.shift_right_logical(i, 1)
    y = lax.bitcast_convert_type(i, jnp.float32)
    for _ in range(3):
        y = y * (1.5 - 0.5 * d * y * y)
    return y


def _adj16(v):
    off = jnp.full((16,), PADOFF, jnp.int32)
    return v + jnp.where(v >= NU, off, jnp.zeros((16,), jnp.int32))


@functools.partial(
    pl.kernel,
    out_type=jax.ShapeDtypeStruct((E_PAD,), jnp.float32),
    mesh=_MESH,
    compiler_params=pltpu.CompilerParams(needs_layout_passes=False, use_tc_tiling_on_sc=False),
    scratch_types=[
        pltpu.VMEM((NPAD,), jnp.float32),
        pltpu.VMEM((CH,), jnp.int32),
        pltpu.VMEM((CH,), jnp.int32),
        pltpu.VMEM((CH,), jnp.float32),
        pltpu.VMEM((CH,), jnp.float32),
        pltpu.VMEM((SL,), jnp.float32),
        pltpu.VMEM_SHARED((16 * NPAD,), jnp.float32),
    ],
)
def _prep(row_hbm, col_hbm, w_hbm, norm_hbm, dloc, colb, rowb, wb, nb, tmpb,
          shared):
    c = lax.axis_index("c")
    s = lax.axis_index("s")
    zero16 = jnp.zeros((16,), jnp.float32)

    def zbody(i, carry):
        dloc[pl.ds(i * 16, 16)] = zero16
        return carry

    lax.fori_loop(0, NPAD // 16, zbody, 0)

    base_t = s * ET
    one16 = jnp.ones((16,), jnp.float32)
    for ch in range(8):
        off = base_t + ch * CH
        pltpu.sync_copy(col_hbm.at[pl.ds(off, CH)], colb)

        def dbody(g, carry):
            c16 = colb[pl.ds(g * 16, 16)]
            gi = off + g * 16 + lax.iota(jnp.int32, 16)
            ones = jnp.where(gi < E, one16, zero16)
            plsc.addupdate_scatter(dloc, [_adj16(c16)], ones)
            return carry

        lax.fori_loop(0, CH // 16, dbody, 0)

    pltpu.sync_copy(dloc, shared.at[pl.ds(s * NPAD, NPAD)])
    plsc.subcore_barrier()

    pltpu.sync_copy(shared.at[pl.ds(s * SL, SL)], dloc.at[pl.ds(0, SL)])
    for k in range(1, 16):
        pltpu.sync_copy(shared.at[pl.ds(k * NPAD + s * SL, SL)], tmpb)

        def abody(i, carry):
            sl = pl.ds(i * 16, 16)
            dloc[sl] = dloc[sl] + tmpb[sl]
            return carry

        lax.fori_loop(0, SL // 16, abody, 0)
    plsc.subcore_barrier()

    def rbody(i, carry):
        d = dloc[pl.ds(i * 16, 16)]
        y = _rsqrt16(d)
        dloc[pl.ds(i * 16, 16)] = jnp.where(d > 0.0, y, zero16)
        return carry

    lax.fori_loop(0, SL // 16, rbody, 0)
    pltpu.sync_copy(dloc.at[pl.ds(0, SL)], shared.at[pl.ds(s * SL, SL)])
    plsc.subcore_barrier()
    pltpu.sync_copy(shared.at[pl.ds(0, NPAD)], dloc)

    base_n = (c * 16 + s) * EW
    for ch in range(4):
        off = base_n + ch * CH
        pltpu.sync_copy(row_hbm.at[pl.ds(off, CH)], rowb)
        pltpu.sync_copy(col_hbm.at[pl.ds(off, CH)], colb)
        pltpu.sync_copy(w_hbm.at[pl.ds(off, CH)], wb)

        def nbody(g, carry):
            r16 = rowb[pl.ds(g * 16, 16)]
            c16 = colb[pl.ds(g * 16, 16)]
            a = plsc.load_gather(dloc, [_adj16(r16)])
            b = plsc.load_gather(dloc, [_adj16(c16)])
            nb[pl.ds(g * 16, 16)] = a * b * wb[pl.ds(g * 16, 16)]
            return carry

        lax.fori_loop(0, CH // 16, nbody, 0)
        pltpu.sync_copy(nb, norm_hbm.at[pl.ds(off, CH)])


@functools.partial(
    pl.kernel,
    out_type=jax.ShapeDtypeStruct((NPAD, D), jnp.float32),
    mesh=_MESH,
    compiler_params=pltpu.CompilerParams(needs_layout_passes=False, use_tc_tiling_on_sc=False),
    scratch_types=[
        pltpu.VMEM((K, D), jnp.float32),
        pltpu.VMEM((3, 128), jnp.int32),
        pltpu.VMEM((3, 128), jnp.int32),
        pltpu.VMEM((2, K), jnp.int32),
        pltpu.VMEM((2, K), jnp.int32),
        pltpu.VMEM((2, K), jnp.float32),
        pltpu.VMEM_SHARED((P, D), jnp.float32),
        pltpu.SemaphoreType.DMA,
        pltpu.SemaphoreType.DMA,
        pltpu.SemaphoreType.DMA,
        pltpu.SemaphoreType.DMA,
        pltpu.SemaphoreType.DMA,
        pltpu.SemaphoreType.DMA,
        pltpu.SemaphoreType.DMA,
    ],
)
def _layer(x_hbm, row_hbm, col_hbm, norm_hbm, out_hbm,
           rows, ridx, cidx, rowi, coli, normi, acc,
           isem, gsem0, gsem1, gsem2, ssem0, ssem1, ssem2):
    c = lax.axis_index("c")
    s = lax.axis_index("s")
    gsems = (gsem0, gsem1, gsem2)
    ssems = (ssem0, ssem1, ssem2)
    zero16 = jnp.zeros((16,), jnp.float32)

    def zbody(r, carry):
        for q in range(4):
            rows[r, pl.ds(q * 16, 16)] = zero16
        return carry

    lax.fori_loop(0, K, zbody, 0)
    arow = s * ROWS_T
    for off, sz in ((0, 384), (384, 384), (768, 384), (1152, 384), (1536, 32)):
        pltpu.sync_copy(rows.at[pl.ds(0, sz)], acc.at[pl.ds(arow + off, sz)])
    plsc.subcore_barrier()

    base_t = s * ET
    cP = c * P
    loc_lo = jnp.zeros((16,), jnp.int32)
    loc_hi = jnp.full((16,), P, jnp.int32)

    def _inputs(t, slot):
        off = base_t + t * K
        for src, dst in ((row_hbm, rowi), (col_hbm, coli), (norm_hbm, normi)):
            pltpu.make_async_copy(
                src.at[pl.ds(off, K)], dst.at[slot], isem).start()

    def _inputs_wait(t, slot):
        off = base_t + t * K
        for src, dst in ((row_hbm, rowi), (col_hbm, coli), (norm_hbm, normi)):
            pltpu.make_async_copy(
                src.at[pl.ds(off, K)], dst.at[slot], isem).wait()

    _inputs(0, 0)

    def chunk(t, carry):
        pb = lax.rem(t, 2)
        _inputs_wait(t, pb)
        gds = []
        for j in range(3):

            def gb(g, carry2, j=j):
                bg = j * 128 + g * 16
                r16 = rowi[pb, pl.ds(bg, 16)]
                ridx[j, pl.ds(g * 16, 16)] = _adj16(r16)
                c16 = coli[pb, pl.ds(bg, 16)]
                loc = _adj16(c16) - cP
                m = (loc >= loc_lo) & (loc < loc_hi)
                cidx[j, pl.ds(g * 16, 16)] = jnp.where(m, loc, loc_lo)
                n16 = normi[pb, pl.ds(bg, 16)]
                normi[pb, pl.ds(bg, 16)] = jnp.where(m, n16, zero16)
                return carry2

            lax.fori_loop(0, 8, gb, 0)
            gds.append(pltpu.async_copy(
                x_hbm.at[ridx.at[j]], rows.at[pl.ds(j * 128, 128)], gsems[j]))
        @pl.when(t < NCHUNK - 1)
        def _():
            _inputs(t + 1, 1 - pb)

        sds = []
        for j in range(3):
            gds[j].wait()
            pbK = pb * K

            def sb(r, carry2, j=j, pbK=pbK):
                rr = j * 128 + r
                sp = plsc.load_gather(
                    normi, [lax.broadcast(pb, (16,)),
                            lax.broadcast(rr, (16,))])
                for q in range(4):
                    rows[rr, pl.ds(q * 16, 16)] = (
                        rows[rr, pl.ds(q * 16, 16)] * sp)
                return carry2

            lax.fori_loop(0, 128, sb, 0)
            sds.append(pltpu.async_copy(
                rows.at[pl.ds(j * 128, 128)], acc.at[cidx.at[j]],
                ssems[j], add=True))
        for d in sds:
            d.wait()
        return carry

    lax.fori_loop(0, NCHUNK, chunk, 0)
    plsc.subcore_barrier()
    for off, sz in ((0, 384), (384, 384), (768, 384), (1152, 384), (1536, 32)):
        pltpu.sync_copy(acc.at[pl.ds(arow + off, sz)],
                        out_hbm.at[pl.ds(cP + arow + off, sz)])


def _mean_body(a_ref, b_ref, c_ref, d_ref, o_ref):
    o_ref[...] = 0.25 * (a_ref[...] + b_ref[...] + c_ref[...] + d_ref[...])


_BR = (NPAD * D // 128) // 8
_mean4 = pl.pallas_call(
    _mean_body,
    grid=(8,),
    in_specs=[pl.BlockSpec((_BR, 128), lambda i: (i, 0))] * 4,
    out_specs=pl.BlockSpec((_BR, 128), lambda i: (i, 0)),
    out_shape=jax.ShapeDtypeStruct((NPAD * D // 128, 128), jnp.float32),
)


def kernel(edge_index, edge_weight, user_emb, item_emb):
    row = jnp.pad(edge_index[0], (0, E_PAD - E))
    col = jnp.pad(edge_index[1], (0, E_PAD - E))
    w = jnp.pad(edge_weight, (0, E_PAD - E))
    x0 = jnp.zeros((NPAD, D), jnp.float32)
    x0 = lax.dynamic_update_slice(x0, user_emb, (0, 0))
    x0 = lax.dynamic_update_slice(x0, item_emb, (P, 0))

    norm = _prep(row, col, w)
    x1 = _layer(x0, row, col, norm)
    x2 = _layer(x1, row, col, norm)
    x3 = _layer(x2, row, col, norm)

    flat = lambda a: a.reshape(NPAD * D // 128, 128)
    final = _mean4(flat(x0), flat(x1), flat(x2), flat(x3)).reshape(NPAD, D)
    return final[:NU], final[P:P + NI]


# speedup vs baseline: 9.8991x; 1.1574x over previous
import functools

import jax
import jax.numpy as jnp
from jax import lax
from jax.experimental import pallas as pl
from jax.experimental.pallas import tpu as pltpu
from jax.experimental.pallas import tpu_sc as plsc

NU = 25000
NI = 25000
D = 64
E = 800000
P = 25088
PADOFF = P - NU
NPAD = 2 * P
SL = NPAD // 16
ROWS_T = P // 16
E_PAD = 804864
ET = E_PAD // 16
CH = 2096
K = 384
REG = ET + K
RCAP = 2688
NB = 32 * REG
_MESH = plsc.VectorSubcoreMesh(core_axis_name="c", subcore_axis_name="s")
_PARAMS = pltpu.CompilerParams(
    needs_layout_passes=False, use_tc_tiling_on_sc=False)


def _rsqrt16(d):
    i = lax.bitcast_convert_type(d, jnp.int32)
    i = jnp.full((16,), 0x5F3759DF, jnp.int32) - lax.shift_right_logical(i, 1)
    y = lax.bitcast_convert_type(i, jnp.float32)
    for _ in range(3):
        y = y * (1.5 - 0.5 * d * y * y)
    return y


def _adj16(v):
    off = jnp.full((16,), PADOFF, jnp.int32)
    return v + jnp.where(v >= NU, off, jnp.zeros((16,), jnp.int32))


@functools.partial(
    pl.kernel,
    out_type=(
        jax.ShapeDtypeStruct((NB,), jnp.int32),
        jax.ShapeDtypeStruct((NB,), jnp.int32),
        jax.ShapeDtypeStruct((NB,), jnp.float32),
        jax.ShapeDtypeStruct((512,), jnp.int32),
    ),
    mesh=_MESH,
    compiler_params=_PARAMS,
    scratch_types=[
        pltpu.VMEM((NPAD,), jnp.float32),
        pltpu.VMEM((CH,), jnp.int32),
        pltpu.VMEM((CH,), jnp.int32),
        pltpu.VMEM((CH,), jnp.float32),
        pltpu.VMEM((SL,), jnp.float32),
        pltpu.VMEM((RCAP,), jnp.int32),
        pltpu.VMEM((RCAP,), jnp.int32),
        pltpu.VMEM((RCAP,), jnp.float32),
        pltpu.VMEM((16,), jnp.int32),
        pltpu.VMEM_SHARED((16 * NPAD,), jnp.float32),
    ],
)
def _prep(row_hbm, col_hbm, w_hbm, br_hbm, bc_hbm, bn_hbm, cnt_hbm,
          dloc, rowb, colb, wb, tmpb, ringr, ringc, ringn, ctmp, shared):
    c = lax.axis_index("c")
    s = lax.axis_index("s")
    zero16 = jnp.zeros((16,), jnp.float32)
    zero16i = jnp.zeros((16,), jnp.int32)
    one16 = jnp.ones((16,), jnp.float32)

    def zbody(i, carry):
        dloc[pl.ds(i * 16, 16)] = zero16
        return carry

    lax.fori_loop(0, NPAD // 16, zbody, 0)

    base_t = s * ET
    for ch in range(ET // CH):
        off = base_t + ch * CH
        pltpu.sync_copy(col_hbm.at[pl.ds(off, CH)], colb)

        def dbody(g, carry):
            c16 = colb[pl.ds(g * 16, 16)]
            gi = off + g * 16 + lax.iota(jnp.int32, 16)
            ones = jnp.where(gi < E, one16, zero16)
            plsc.addupdate_scatter(dloc, [_adj16(c16)], ones)
            return carry

        lax.fori_loop(0, CH // 16, dbody, 0)

    pltpu.sync_copy(dloc, shared.at[pl.ds(s * NPAD, NPAD)])
    plsc.subcore_barrier()

    pltpu.sync_copy(shared.at[pl.ds(s * SL, SL)], dloc.at[pl.ds(0, SL)])
    for k in range(1, 16):
        pltpu.sync_copy(shared.at[pl.ds(k * NPAD + s * SL, SL)], tmpb)

        def abody(i, carry):
            sl = pl.ds(i * 16, 16)
            dloc[sl] = dloc[sl] + tmpb[sl]
            return carry

        lax.fori_loop(0, SL // 16, abody, 0)
    plsc.subcore_barrier()

    def rbody(i, carry):
        d = dloc[pl.ds(i * 16, 16)]
        y = _rsqrt16(d)
        dloc[pl.ds(i * 16, 16)] = jnp.where(d > 0.0, y, zero16)
        return carry

    lax.fori_loop(0, SL // 16, rbody, 0)
    pltpu.sync_copy(dloc.at[pl.ds(0, SL)], shared.at[pl.ds(s * SL, SL)])
    plsc.subcore_barrier()
    pltpu.sync_copy(shared.at[pl.ds(0, NPAD)], dloc)

    def zring(i, carry):
        sl = pl.ds(i * 16, 16)
        ringr[sl] = zero16i
        ringc[sl] = zero16i
        ringn[sl] = zero16
        return carry

    lax.fori_loop(0, RCAP // 16, zring, 0)

    cP = c * P
    wid = c * 16 + s
    base_o = wid * REG
    loc_lo = jnp.zeros((16,), jnp.int32)
    loc_hi = jnp.full((16,), P, jnp.int32)
    one16i = jnp.ones((16,), jnp.int32)
    iota16 = lax.iota(jnp.int32, 16)
    lane15 = jnp.full((16, 1), 15, jnp.int32)
    _dn = lax.GatherDimensionNumbers(
        offset_dims=(), collapsed_slice_dims=(0,), start_index_map=(0,))
    bvec = jnp.zeros((16,), jnp.int32)
    blk = jnp.int32(0)
    for ch in range(ET // CH):
        off = base_t + ch * CH
        pltpu.sync_copy(row_hbm.at[pl.ds(off, CH)], rowb)
        pltpu.sync_copy(col_hbm.at[pl.ds(off, CH)], colb)
        pltpu.sync_copy(w_hbm.at[pl.ds(off, CH)], wb)

        def pbody(g, bcur):
            sl = pl.ds(g * 16, 16)
            radj = _adj16(rowb[sl])
            cadj = _adj16(colb[sl])
            loc = cadj - cP
            m = (loc >= loc_lo) & (loc < loc_hi)
            a = plsc.load_gather(dloc, [radj])
            bb = plsc.load_gather(dloc, [cadj])
            nrm = a * bb * wb[sl]
            pc = plsc.cumsum(jnp.where(m, one16i, jnp.zeros((16,), jnp.int32)))
            dest = bcur + pc - one16i
            plsc.store_scatter(ringr, [dest], radj, mask=m)
            plsc.store_scatter(ringc, [dest], loc, mask=m)
            plsc.store_scatter(ringn, [dest], nrm, mask=m)
            tot = lax.gather(pc, lane15, _dn, (1,),
                             mode=lax.GatherScatterMode.PROMISE_IN_BOUNDS)
            return bcur + tot

        bvec = lax.fori_loop(0, CH // 16, pbody, bvec)
        b = jnp.max(bvec)
        nfl = lax.div(b, jnp.int32(K))

        def fbody(i, carry):
            src = pl.ds(i * K, K)
            dst = pl.ds(base_o + (blk + i) * K, K)
            pltpu.sync_copy(ringr.at[src], br_hbm.at[dst])
            pltpu.sync_copy(ringc.at[src], bc_hbm.at[dst])
            pltpu.sync_copy(ringn.at[src], bn_hbm.at[dst])
            return carry

        lax.fori_loop(0, nfl, fbody, 0)
        blk = blk + nfl
        b = b - nfl * K
        bvec = lax.broadcast(b, (16,))

        def mbody(q, carry, base=nfl * K):
            sidx = base + q * 16 + iota16
            didx = q * 16 + iota16
            plsc.store_scatter(ringr, [didx], plsc.load_gather(ringr, [sidx]))
            plsc.store_scatter(ringc, [didx], plsc.load_gather(ringc, [sidx]))
            plsc.store_scatter(ringn, [didx], plsc.load_gather(ringn, [sidx]))
            return carry

        lax.fori_loop(0, K // 16, mbody, 0)

    b = jnp.max(bvec)
    def tzero(q, carry):
        plsc.store_scatter(ringn, [b + q * 16 + iota16], zero16)
        return carry

    lax.fori_loop(0, K // 16, tzero, 0)
    src = pl.ds(0, K)
    dst = pl.ds(base_o + blk * K, K)
    pltpu.sync_copy(ringr.at[src], br_hbm.at[dst])
    pltpu.sync_copy(ringc.at[src], bc_hbm.at[dst])
    pltpu.sync_copy(ringn.at[src], bn_hbm.at[dst])
    ctmp[pl.ds(0, 16)] = lax.broadcast(blk + 1, (16,))
    pltpu.sync_copy(ctmp, cnt_hbm.at[pl.ds(wid * 16, 16)])


@functools.partial(
    pl.kernel,
    out_type=jax.ShapeDtypeStruct((NPAD, D), jnp.float32),
    mesh=_MESH,
    compiler_params=_PARAMS,
    scratch_types=[
        pltpu.VMEM((K, D), jnp.float32),
        pltpu.VMEM((3, 128), jnp.int32),
        pltpu.VMEM((2, 3, 128), jnp.int32),
        pltpu.VMEM((2, K), jnp.int32),
        pltpu.VMEM((2, K), jnp.int32),
        pltpu.VMEM((2, K), jnp.float32),
        pltpu.VMEM((16,), jnp.int32),
        pltpu.VMEM_SHARED((P, D), jnp.float32),
        pltpu.SemaphoreType.DMA,
        pltpu.SemaphoreType.DMA,
        pltpu.SemaphoreType.DMA,
        pltpu.SemaphoreType.DMA,
        pltpu.SemaphoreType.DMA,
        pltpu.SemaphoreType.DMA,
        pltpu.SemaphoreType.DMA,
    ],
)
def _layer(x_hbm, br_hbm, bc_hbm, bn_hbm, cnt_hbm, out_hbm,
           rows, ridx, cidx, rin, cin, nin, cbuf, acc,
           isem, gsem0, gsem1, gsem2, ssem0, ssem1, ssem2):
    c = lax.axis_index("c")
    s = lax.axis_index("s")
    gsems = (gsem0, gsem1, gsem2)
    ssems = (ssem0, ssem1, ssem2)
    zero16 = jnp.zeros((16,), jnp.float32)

    def zbody(r, carry):
        for q in range(4):
            rows[r, pl.ds(q * 16, 16)] = zero16
        return carry

    lax.fori_loop(0, K, zbody, 0)
    arow = s * ROWS_T
    for off, sz in ((0, 384), (384, 384), (768, 384), (1152, 384), (1536, 32)):
        pltpu.sync_copy(rows.at[pl.ds(0, sz)], acc.at[pl.ds(arow + off, sz)])

    wid = c * 16 + s
    base_o = wid * REG
    pltpu.sync_copy(cnt_hbm.at[pl.ds(wid * 16, 16)], cbuf)
    nch = jnp.max(cbuf[pl.ds(0, 16)])
    plsc.subcore_barrier()

    def _inputs(t, slot):
        off = base_o + t * K
        for src, dstb in ((br_hbm, rin), (bc_hbm, cin), (bn_hbm, nin)):
            pltpu.make_async_copy(
                src.at[pl.ds(off, K)], dstb.at[slot], isem).start()

    def _inputs_wait(t, slot):
        off = base_o + t * K
        for src, dstb in ((br_hbm, rin), (bc_hbm, cin), (bn_hbm, nin)):
            pltpu.make_async_copy(
                src.at[pl.ds(off, K)], dstb.at[slot], isem).wait()

    _inputs(0, 0)

    def chunk(t, carry):
        pb = lax.rem(t, 2)
        _inputs_wait(t, pb)
        for j in range(3):

            def cc(g, carry2, j=j):
                sl = pl.ds(j * 128 + g * 16, 16)
                ridx[j, pl.ds(g * 16, 16)] = rin[pb, sl]
                cidx[pb, j, pl.ds(g * 16, 16)] = cin[pb, sl]
                return carry2

            lax.fori_loop(0, 8, cc, 0)
        @pl.when(t > 0)
        def _():
            for j in range(3):
                pltpu.make_async_copy(
                    rows.at[pl.ds(j * 128, 128)],
                    acc.at[cidx.at[1 - pb, j]], ssems[j]).wait()

        for j in range(3):
            pltpu.async_copy(
                x_hbm.at[ridx.at[j]], rows.at[pl.ds(j * 128, 128)], gsems[j])

        @pl.when(t < nch - 1)
        def _():
            _inputs(t + 1, 1 - pb)

        for j in range(3):
            pltpu.make_async_copy(
                x_hbm.at[ridx.at[j]], rows.at[pl.ds(j * 128, 128)],
                gsems[j]).wait()

            def sb(g, carry2, j=j):
                sp16 = nin[pb, pl.ds(j * 128 + g * 16, 16)]
                rr0 = j * 128 + g * 16
                dn = lax.GatherDimensionNumbers(
                    offset_dims=(), collapsed_slice_dims=(0,),
                    start_index_map=(0,))
                for u in range(16):
                    spl = lax.gather(
                        sp16, jnp.full((16, 1), u, jnp.int32), dn, (1,),
                        mode=lax.GatherScatterMode.PROMISE_IN_BOUNDS)
                    for q in range(4):
                        rows[rr0 + u, pl.ds(q * 16, 16)] = (
                            rows[rr0 + u, pl.ds(q * 16, 16)] * spl)
                return carry2

            lax.fori_loop(0, 8, sb, 0)
            pltpu.async_copy(
                rows.at[pl.ds(j * 128, 128)], acc.at[cidx.at[pb, j]],
                ssems[j], add=True)
        return carry

    lax.fori_loop(0, nch, chunk, 0)
    lastpb = lax.rem(nch - 1, 2)
    for j in range(3):
        pltpu.make_async_copy(
            rows.at[pl.ds(j * 128, 128)],
            acc.at[cidx.at[lastpb, j]], ssems[j]).wait()
    plsc.subcore_barrier()
    for off, sz in ((0, 384), (384, 384), (768, 384), (1152, 384), (1536, 32)):
        pltpu.sync_copy(acc.at[pl.ds(arow + off, sz)],
                        out_hbm.at[pl.ds(c * P + arow + off, sz)])


def _mean_body(a_ref, b_ref, c_ref, d_ref, o_ref):
    o_ref[...] = 0.25 * (a_ref[...] + b_ref[...] + c_ref[...] + d_ref[...])


_BR = (NPAD * D // 128) // 8
_mean4 = pl.pallas_call(
    _mean_body,
    grid=(8,),
    in_specs=[pl.BlockSpec((_BR, 128), lambda i: (i, 0))] * 4,
    out_specs=pl.BlockSpec((_BR, 128), lambda i: (i, 0)),
    out_shape=jax.ShapeDtypeStruct((NPAD * D // 128, 128), jnp.float32),
)


def kernel(edge_index, edge_weight, user_emb, item_emb):
    row = jnp.pad(edge_index[0], (0, E_PAD - E))
    col = jnp.pad(edge_index[1], (0, E_PAD - E))
    w = jnp.pad(edge_weight, (0, E_PAD - E))
    x0 = jnp.zeros((NPAD, D), jnp.float32)
    x0 = lax.dynamic_update_slice(x0, user_emb, (0, 0))
    x0 = lax.dynamic_update_slice(x0, item_emb, (P, 0))

    br, bc, bn, cnt = _prep(row, col, w)
    x1 = _layer(x0, br, bc, bn, cnt)
    x2 = _layer(x1, br, bc, bn, cnt)
    x3 = _layer(x2, br, bc, bn, cnt)

    flat = lambda a: a.reshape(NPAD * D // 128, 128)
    final = _mean4(flat(x0), flat(x1), flat(x2), flat(x3)).reshape(NPAD, D)
    return final[:NU], final[P:P + NI]
